# baseline (speedup 1.0000x reference)
import jax
import jax.numpy as jnp
from jax import lax
from jax.experimental import pallas as pl
from jax.experimental.pallas import tpu as pltpu
from jax.experimental.pallas import tpu_sc as plsc

NC = 2
NS = 16
NW = NC * NS
CH = 128
G = 20

V = 1000000
D = 32
NBLK = (V + 127) // 128


def _lin_body(src, tail, dst):
    is_last = pl.program_id(0) == NBLK - 1
    blk = jnp.where(is_last, tail[...], src[...])
    iot_i = lax.broadcasted_iota(jnp.int32, (32, 128), 0)
    iot_l = lax.broadcasted_iota(jnp.int32, (32, 128), 1)
    parts = []
    for a in range(4):
        sel = (iot_l == 4 * iot_i + a).astype(jnp.float32)
        parts.append(
            lax.dot_general(sel, blk, (((1,), (1,)), ((), ())),
                            preferred_element_type=jnp.float32)
        )
    dst[...] = jnp.concatenate(parts, axis=1)


def _table_to_linear(lut):
    tail = jnp.pad(lut[(NBLK - 1) * 128:], ((0, NBLK * 128 - V), (0, 0))).T
    return pl.pallas_call(
        _lin_body,
        grid=(NBLK,),
        in_specs=[
            pl.BlockSpec((D, 128), lambda c: (0, jnp.minimum(c, NBLK - 2))),
            pl.BlockSpec((D, 128), lambda c: (0, 0)),
        ],
        out_specs=pl.BlockSpec((D, 128), lambda c: (c, 0)),
        out_shape=jax.ShapeDtypeStruct((NBLK * D, 128), jnp.float32),
    )(lut.T, tail)


def _emb_body(idx_hbm, table_hbm, out_hbm, idx_v, rows_v, sem):
    n_chunks = idx_hbm.shape[0] // CH
    chunks_per_w = n_chunks // NW
    wid = lax.axis_index("s") * NC + lax.axis_index("c")
    row0 = wid * chunks_per_w

    def outer(g, carry):
        base = (row0 + g * G) * CH
        pltpu.sync_copy(idx_hbm.at[pl.ds(base, G * CH)], idx_v)
        copies = [
            pltpu.async_copy(
                table_hbm.at[idx_v.at[pl.ds(j * CH, CH)]],
                rows_v.at[pl.ds(j * CH, CH)],
                sem,
            )
            for j in range(G)
        ]
        for cp in copies:
            cp.wait()
        pltpu.sync_copy(rows_v, out_hbm.at[pl.ds(base, G * CH)])
        return carry

    lax.fori_loop(0, chunks_per_w // G, outer, 0)


def _gather(idx_flat, table):
    mesh = plsc.VectorSubcoreMesh(core_axis_name="c", subcore_axis_name="s")
    b = idx_flat.shape[0]
    k = pl.kernel(
        _emb_body,
        mesh=mesh,
        out_type=jax.ShapeDtypeStruct((b, D), jnp.float32),
        compiler_params=pltpu.CompilerParams(use_tc_tiling_on_sc=False),
        scratch_types=[
            pltpu.VMEM((G * CH,), jnp.int32),
            pltpu.VMEM((G * CH, D), jnp.float32),
            pltpu.SemaphoreType.DMA,
        ],
    )
    return k(idx_flat, table)


def _out_body(src, dst):
    blk = src[...]
    iot_m = lax.broadcasted_iota(jnp.int32, (32, 128), 0)
    iot_l = lax.broadcasted_iota(jnp.int32, (32, 128), 1)
    acc = jnp.zeros((32, 128), jnp.float32)
    for r in range(4):
        part = blk[:, 32 * r:32 * r + 32]
        g = (iot_l == 4 * iot_m + r).astype(jnp.float32)
        acc = acc + lax.dot_general(part, g, (((0,), (0,)), ((), ())),
                                    preferred_element_type=jnp.float32)
    dst[...] = acc[None]


def _rows_to_final(g128):
    return pl.pallas_call(
        _out_body,
        grid=(50, 128),
        in_specs=[pl.BlockSpec((32, 128), lambda s, nb: (s * 128 + nb, 0))],
        out_specs=pl.BlockSpec((1, 32, 128), lambda s, nb: (s, 0, nb)),
        out_shape=jax.ShapeDtypeStruct((50, 32, 16384), jnp.float32),
    )(g128)


def kernel(token_ids, lut):
    n, s = token_ids.shape
    b = n * s
    idx_bt = token_ids.T.reshape(b).astype(jnp.int32)
    lin = _table_to_linear(lut)
    table = lin.reshape(NBLK * 128, D)
    g = _gather(idx_bt, table)
    g128 = g.reshape(b * D // 128, 128)
    out_p = _rows_to_final(g128)
    return jnp.transpose(out_p, (2, 0, 1))


# speedup vs baseline: 4.6086x; 4.6086x over previous
import jax
import jax.numpy as jnp
from jax import lax
from jax.experimental import pallas as pl
from jax.experimental.pallas import tpu as pltpu
from jax.experimental.pallas import tpu_sc as plsc

NC = 2
NS = 16
NW = NC * NS

V = 1000000
D = 32
NBLK = (V + 127) // 128
VP = NBLK * 128

N_TOK = 16384
S_TOK = 50
B = N_TOK * S_TOK
NOUT = B // 128
K1 = 8


def _b0_body(lut_t, tail, table, in_v, out_v):
    wid = lax.axis_index("s") * NC + lax.axis_index("c")
    n_c = NBLK // NW + jnp.where(wid < NBLK % NW, 1, 0)
    start = wid * (NBLK // NW) + jnp.minimum(wid, NBLK % NW)

    iota16 = lax.broadcasted_iota(jnp.int32, (16,), 0)

    def per_tile(t, carry):
        c = start + t
        is_last = c == NBLK - 1

        @pl.when(jnp.logical_not(is_last))
        def _():
            pltpu.sync_copy(lut_t.at[:, pl.ds(c * 128, 128)], in_v)

        @pl.when(is_last)
        def _():
            pltpu.sync_copy(tail, in_v)

        for i in range(32):
            for t8 in range(8):
                j0 = 16 * t8
                idx_f = (j0 % 32) + iota16
                idx_l = jnp.full((16,), 4 * i + j0 // 32, jnp.int32)
                out_v[i, pl.ds(j0, 16)] = plsc.load_gather(in_v, [idx_f, idx_l])
        pltpu.sync_copy(out_v, table.at[pl.ds(c * 32, 32)])
        return carry

    lax.fori_loop(0, n_c, per_tile, 0)


def _relayout(lut):
    tail = jnp.pad(lut[(NBLK - 1) * 128:], ((0, VP - V), (0, 0))).T
    mesh = plsc.VectorSubcoreMesh(core_axis_name="c", subcore_axis_name="s")
    k = pl.kernel(
        _b0_body,
        mesh=mesh,
        out_type=jax.ShapeDtypeStruct((NBLK * 32, 128), jnp.float32),
        compiler_params=pltpu.CompilerParams(needs_layout_passes=False),
        scratch_types=[
            pltpu.VMEM((32, 128), jnp.float32),
            pltpu.VMEM((32, 128), jnp.float32),
        ],
    )
    return k(lut.T, tail)


def _b1_body(idx_hbm, table, out5, idx_v, rows_v, trans_v, sem_g, sem_w):
    wid = lax.axis_index("s") * NC + lax.axis_index("c")
    blocks_per_w = NOUT // NW
    b0 = wid * blocks_per_w

    iota16 = lax.broadcasted_iota(jnp.int32, (16,), 0)

    def per_batch(g, carry):
        base_blk = b0 + g * K1
        pltpu.sync_copy(idx_hbm.at[pl.ds(base_blk * 128, K1 * 128)], idx_v)
        copies = [
            pltpu.async_copy(
                table.at[idx_v.at[pl.ds(k * 128, 128)]],
                rows_v.at[pl.ds(k * 128, 128)],
                sem_g,
            )
            for k in range(K1)
        ]
        for cp in copies:
            cp.wait()

        def per_block(k, carry2):
            blk = base_blk + k
            s = blk // 128
            nb = blk % 128
            for dg in range(4):
                for d8 in range(8):
                    f = 8 * dg + d8
                    for t8 in range(8):
                        n0 = 16 * t8
                        idx_n = k * 128 + n0 + iota16
                        idx_f = jnp.full((16,), f, jnp.int32)
                        trans_v[k, dg, d8, pl.ds(n0, 16)] = plsc.load_gather(
                            rows_v, [idx_n, idx_f]
                        )
            pltpu.async_copy(trans_v.at[k], out5.at[s, :, nb], sem_w)
            return carry2

        lax.fori_loop(0, K1, per_block, 0)
        for _ in range(K1):
            pltpu.make_async_copy(trans_v.at[0], out5.at[0, :, 0], sem_w).wait()
        return carry

    lax.fori_loop(0, blocks_per_w // K1, per_batch, 0)


def _lookup(idx_flat, table):
    mesh = plsc.VectorSubcoreMesh(core_axis_name="c", subcore_axis_name="s")
    k = pl.kernel(
        _b1_body,
        mesh=mesh,
        out_type=jax.ShapeDtypeStruct((S_TOK, 4, 128, 8, 128), jnp.float32),
        compiler_params=pltpu.CompilerParams(
            use_tc_tiling_on_sc=False, needs_layout_passes=False
        ),
        scratch_types=[
            pltpu.VMEM((K1 * 128,), jnp.int32),
            pltpu.VMEM((K1 * 128, D), jnp.float32),
            pltpu.VMEM((K1, 4, 8, 128), jnp.float32),
            pltpu.SemaphoreType.DMA,
            pltpu.SemaphoreType.DMA,
        ],
    )
    return k(idx_flat, table)


def kernel(token_ids, lut):
    idx_bt = token_ids.T.reshape(B).astype(jnp.int32)
    lin = _relayout(lut)
    table = lin.reshape(VP, D)
    out5 = _lookup(idx_bt, table)
    return jnp.transpose(out5, (2, 4, 0, 1, 3)).reshape(N_TOK, S_TOK, D)


# speedup vs baseline: 5.0417x; 1.0940x over previous
import jax
import jax.numpy as jnp
from jax import lax
from jax.experimental import pallas as pl
from jax.experimental.pallas import tpu as pltpu
from jax.experimental.pallas import tpu_sc as plsc

NC = 2
NS = 16
NW = NC * NS

V = 1000000
D = 32
NBLK = (V + 127) // 128
VP = NBLK * 128

N_TOK = 16384
S_TOK = 50
B = N_TOK * S_TOK
NOUT = B // 128
K1 = 8


TB = 8
B0_FULL = (NBLK // NW) // TB


def _b0_body(lut_t, tail, table, in_v, out_v):
    wid = lax.axis_index("s") * NC + lax.axis_index("c")
    n_c = NBLK // NW + jnp.where(wid < NBLK % NW, 1, 0)
    start = wid * (NBLK // NW) + jnp.minimum(wid, NBLK % NW)

    iota16 = lax.broadcasted_iota(jnp.int32, (16,), 0)
    idx_f = [j0 % 32 + iota16 for j0 in (0, 16)]

    def permute_tile(kk, lane0, row0):
        def per_i(i, carry):
            for u in range(4):
                idx_l = jnp.full((16,), lane0 + 4 * i + u, jnp.int32)
                for par in range(2):
                    j0 = 32 * u + 16 * par
                    out_v[row0 + i, pl.ds(j0, 16)] = plsc.load_gather(
                        in_v, [idx_f[par], idx_l]
                    )
            return carry
        lax.fori_loop(0, 32, per_i, 0)

    def per_batch(g, carry):
        c0 = start + g * TB
        pltpu.sync_copy(lut_t.at[:, pl.ds(c0 * 128, TB * 128)], in_v)

        def per_k(kk, carry2):
            permute_tile(kk, kk * 128, kk * 32)
            return carry2
        lax.fori_loop(0, TB, per_k, 0)
        pltpu.sync_copy(out_v, table.at[pl.ds(c0 * 32, TB * 32)])
        return carry

    lax.fori_loop(0, B0_FULL, per_batch, 0)

    def per_tile(t, carry):
        c = start + B0_FULL * TB + t
        is_last = c == NBLK - 1

        @pl.when(jnp.logical_not(is_last))
        def _():
            pltpu.sync_copy(lut_t.at[:, pl.ds(c * 128, 128)], in_v.at[:, pl.ds(0, 128)])

        @pl.when(is_last)
        def _():
            pltpu.sync_copy(tail, in_v.at[:, pl.ds(0, 128)])

        permute_tile(0, 0, 0)
        pltpu.sync_copy(out_v.at[pl.ds(0, 32)], table.at[pl.ds(c * 32, 32)])
        return carry

    lax.fori_loop(0, n_c - B0_FULL * TB, per_tile, 0)


def _relayout(lut):
    tail = jnp.pad(lut[(NBLK - 1) * 128:], ((0, VP - V), (0, 0))).T
    mesh = plsc.VectorSubcoreMesh(core_axis_name="c", subcore_axis_name="s")
    k = pl.kernel(
        _b0_body,
        mesh=mesh,
        out_type=jax.ShapeDtypeStruct((NBLK * 32, 128), jnp.float32),
        compiler_params=pltpu.CompilerParams(needs_layout_passes=False),
        scratch_types=[
            pltpu.VMEM((32, TB * 128), jnp.float32),
            pltpu.VMEM((TB * 32, 128), jnp.float32),
        ],
    )
    return k(lut.T, tail)


def _b1_body(idx_hbm, table, out5, idx_v, rows_v, trans_v, sem_g, sem_w):
    wid = lax.axis_index("s") * NC + lax.axis_index("c")
    blocks_per_w = NOUT // NW
    b0 = wid * blocks_per_w

    iota16 = lax.broadcasted_iota(jnp.int32, (16,), 0)

    def per_batch(g, carry):
        base_blk = b0 + g * K1
        pltpu.sync_copy(idx_hbm.at[pl.ds(base_blk * 128, K1 * 128)], idx_v)
        copies = [
            pltpu.async_copy(
                table.at[idx_v.at[pl.ds(k * 128, 128)]],
                rows_v.at[pl.ds(k * 128, 128)],
                sem_g,
            )
            for k in range(K1)
        ]
        for cp in copies:
            cp.wait()

        idx_f = [jnp.full((16,), f, jnp.int32) for f in range(D)]

        def per_block(k, carry2):
            blk = base_blk + k
            s = blk // 128
            nb = blk % 128
            for t8 in range(8):
                n0 = 16 * t8
                idx_n = k * 128 + n0 + iota16
                for f in range(D):
                    trans_v[k, f // 8, f % 8, pl.ds(n0, 16)] = plsc.load_gather(
                        rows_v, [idx_n, idx_f[f]]
                    )
            pltpu.async_copy(trans_v.at[k], out5.at[s, :, nb], sem_w)
            return carry2

        lax.fori_loop(0, K1, per_block, 0)
        for _ in range(K1):
            pltpu.make_async_copy(trans_v.at[0], out5.at[0, :, 0], sem_w).wait()
        return carry

    lax.fori_loop(0, blocks_per_w // K1, per_batch, 0)


def _lookup(idx_flat, table):
    mesh = plsc.VectorSubcoreMesh(core_axis_name="c", subcore_axis_name="s")
    k = pl.kernel(
        _b1_body,
        mesh=mesh,
        out_type=jax.ShapeDtypeStruct((S_TOK, 4, 128, 8, 128), jnp.float32),
        compiler_params=pltpu.CompilerParams(
            use_tc_tiling_on_sc=False, needs_layout_passes=False
        ),
        scratch_types=[
            pltpu.VMEM((K1 * 128,), jnp.int32),
            pltpu.VMEM((K1 * 128, D), jnp.float32),
            pltpu.VMEM((K1, 4, 8, 128), jnp.float32),
            pltpu.SemaphoreType.DMA,
            pltpu.SemaphoreType.DMA,
        ],
    )
    return k(idx_flat, table)


def kernel(token_ids, lut):
    idx_bt = token_ids.T.reshape(B).astype(jnp.int32)
    lin = _relayout(lut)
    table = lin.reshape(VP, D)
    out5 = _lookup(idx_bt, table)
    return jnp.transpose(out5, (2, 4, 0, 1, 3)).reshape(N_TOK, S_TOK, D)


# speedup vs baseline: 7.5551x; 1.4985x over previous
import jax
import jax.numpy as jnp
from jax import lax
from jax.experimental import pallas as pl
from jax.experimental.pallas import tpu as pltpu
from jax.experimental.pallas import tpu_sc as plsc

NC = 2
NS = 16
NW = NC * NS

V = 1000000
D = 32
NBLK = (V + 127) // 128
VP = NBLK * 128

N_TOK = 16384
S_TOK = 50
B = N_TOK * S_TOK
NOUT = B // 128
K1 = 8


TB = 8
B0_FULL = (NBLK // NW) // TB


def _b0_body(lut_t, tail, table, in_v, out_v):
    wid = lax.axis_index("s") * NC + lax.axis_index("c")
    n_c = NBLK // NW + jnp.where(wid < NBLK % NW, 1, 0)
    start = wid * (NBLK // NW) + jnp.minimum(wid, NBLK % NW)

    iota16 = lax.broadcasted_iota(jnp.int32, (16,), 0)
    idx_f = [j0 % 32 + iota16 for j0 in (0, 16)]

    def permute_tile(kk, lane0, row0):
        def per_i(i, carry):
            vals = []
            for u in range(4):
                idx_l = jnp.full((16,), lane0 + 4 * i + u, jnp.int32)
                for par in range(2):
                    vals.append(plsc.load_gather(in_v, [idx_f[par], idx_l]))
            for t8, v in enumerate(vals):
                out_v[row0 + i, pl.ds(16 * t8, 16)] = v
            return carry
        lax.fori_loop(0, 32, per_i, 0)

    def per_batch(g, carry):
        c0 = start + g * TB
        pltpu.sync_copy(lut_t.at[:, pl.ds(c0 * 128, TB * 128)], in_v)

        def per_k(kk, carry2):
            permute_tile(kk, kk * 128, kk * 32)
            return carry2
        lax.fori_loop(0, TB, per_k, 0)
        pltpu.sync_copy(out_v, table.at[pl.ds(c0 * 32, TB * 32)])
        return carry

    lax.fori_loop(0, B0_FULL, per_batch, 0)

    def per_tile(t, carry):
        c = start + B0_FULL * TB + t
        is_last = c == NBLK - 1

        @pl.when(jnp.logical_not(is_last))
        def _():
            pltpu.sync_copy(lut_t.at[:, pl.ds(c * 128, 128)], in_v.at[:, pl.ds(0, 128)])

        @pl.when(is_last)
        def _():
            pltpu.sync_copy(tail, in_v.at[:, pl.ds(0, 128)])

        permute_tile(0, 0, 0)
        pltpu.sync_copy(out_v.at[pl.ds(0, 32)], table.at[pl.ds(c * 32, 32)])
        return carry

    lax.fori_loop(0, n_c - B0_FULL * TB, per_tile, 0)


def _relayout(lut):
    tail = jnp.pad(lut[(NBLK - 1) * 128:], ((0, VP - V), (0, 0))).T
    mesh = plsc.VectorSubcoreMesh(core_axis_name="c", subcore_axis_name="s")
    k = pl.kernel(
        _b0_body,
        mesh=mesh,
        out_type=jax.ShapeDtypeStruct((NBLK * 32, 128), jnp.float32),
        compiler_params=pltpu.CompilerParams(needs_layout_passes=False),
        scratch_types=[
            pltpu.VMEM((32, TB * 128), jnp.float32),
            pltpu.VMEM((TB * 32, 128), jnp.float32),
        ],
    )
    return k(lut.T, tail)


def _b1_body(idx_hbm, table, out5, idx_v, rows_v, trans_v, sem_g, sem_w):
    wid = lax.axis_index("s") * NC + lax.axis_index("c")
    blocks_per_w = NOUT // NW
    b0 = wid * blocks_per_w

    iota16 = lax.broadcasted_iota(jnp.int32, (16,), 0)

    def per_batch(g, carry):
        base_blk = b0 + g * K1
        pltpu.sync_copy(idx_hbm.at[pl.ds(base_blk * 128, K1 * 128)], idx_v)
        copies = [
            pltpu.async_copy(
                table.at[idx_v.at[pl.ds(k * 128, 128)]],
                rows_v.at[pl.ds(k * 128, 128)],
                sem_g,
            )
            for k in range(K1)
        ]
        for cp in copies:
            cp.wait()

        idx_f = [jnp.full((16,), f, jnp.int32) for f in range(D)]

        def per_block(k, carry2):
            blk = base_blk + k
            s = blk // 128
            nb = blk % 128
            for t8 in range(8):
                n0 = 16 * t8
                idx_n = k * 128 + n0 + iota16
                for f8 in range(4):
                    vals = [
                        plsc.load_gather(rows_v, [idx_n, idx_f[8 * f8 + d8]])
                        for d8 in range(8)
                    ]
                    for d8, v in enumerate(vals):
                        f = 8 * f8 + d8
                        trans_v[k, f // 8, f % 8, pl.ds(n0, 16)] = v
            pltpu.async_copy(trans_v.at[k], out5.at[s, :, nb], sem_w)
            return carry2

        lax.fori_loop(0, K1, per_block, 0)
        for _ in range(K1):
            pltpu.make_async_copy(trans_v.at[0], out5.at[0, :, 0], sem_w).wait()
        return carry

    lax.fori_loop(0, blocks_per_w // K1, per_batch, 0)


def _lookup(idx_flat, table):
    mesh = plsc.VectorSubcoreMesh(core_axis_name="c", subcore_axis_name="s")
    k = pl.kernel(
        _b1_body,
        mesh=mesh,
        out_type=jax.ShapeDtypeStruct((S_TOK, 4, 128, 8, 128), jnp.float32),
        compiler_params=pltpu.CompilerParams(
            use_tc_tiling_on_sc=False, needs_layout_passes=False
        ),
        scratch_types=[
            pltpu.VMEM((K1 * 128,), jnp.int32),
            pltpu.VMEM((K1 * 128, D), jnp.float32),
            pltpu.VMEM((K1, 4, 8, 128), jnp.float32),
            pltpu.SemaphoreType.DMA,
            pltpu.SemaphoreType.DMA,
        ],
    )
    return k(idx_flat, table)


def kernel(token_ids, lut):
    idx_bt = token_ids.T.reshape(B).astype(jnp.int32)
    lin = _relayout(lut)
    table = lin.reshape(VP, D)
    out5 = _lookup(idx_bt, table)
    return jnp.transpose(out5, (2, 4, 0, 1, 3)).reshape(N_TOK, S_TOK, D)
